# baseline (speedup 1.0000x reference)
import jax
import jax.numpy as jnp
from jax.experimental import pallas as pl
from jax.experimental.pallas import tpu as pltpu

N = 8192
D = 64
NBINS = 64
MAX_DIST = 2.0
R = 256
C = 512
NRB = N // R
NCB = N // C
NEG_INF = float("-inf")
POS_INF = float("inf")


def _body(rlmin_ref, rlmax_ref, clmin_ref, clmax_ref,
          xb_ref, xtb_ref, tcol_ref, trow_ref,
          out_ref, accp_ref, accn_ref, hv_ref):
    i = pl.program_id(0)
    j = pl.program_id(1)

    @pl.when(j == 0)
    def _init():
        accp_ref[...] = jnp.full((1, R), NEG_INF, jnp.float32)
        accn_ref[...] = jnp.full((1, R), POS_INF, jnp.float32)

    xb = xb_ref[...]
    xtb = xtb_ref[...]
    dot = jax.lax.dot_general(
        xb.astype(jnp.bfloat16), xtb.astype(jnp.bfloat16),
        (((1,), (0,)), ((), ())),
        preferred_element_type=jnp.float32,
    )
    h = 0.5 * jnp.sum(xtb * xtb, axis=0, keepdims=True)
    e = h - dot

    overlap = jnp.logical_and(clmax_ref[j] >= rlmin_ref[i],
                              clmin_ref[j] <= rlmax_ref[i])

    @pl.when(overlap)
    def _band():
        mask = tcol_ref[...] == trow_ref[...]
        pmax = jnp.max(jnp.where(mask, e, NEG_INF), axis=1)
        nmin = jnp.min(jnp.where(mask, POS_INF, e), axis=1)
        accp_ref[...] = jnp.maximum(accp_ref[...], pmax.reshape(1, R))
        accn_ref[...] = jnp.minimum(accn_ref[...], nmin.reshape(1, R))

    @pl.when(jnp.logical_not(overlap))
    def _off_band():
        nmin = jnp.min(e, axis=1)
        accn_ref[...] = jnp.minimum(accn_ref[...], nmin.reshape(1, R))

    @pl.when(j == NCB - 1)
    def _finish_row():
        sq_r = jnp.sum(xb * xb, axis=1).reshape(1, R)
        pos = jnp.sqrt(jnp.clip(sq_r + 2.0 * accp_ref[...], 1e-12, None))
        neg = jnp.sqrt(jnp.clip(sq_r + 2.0 * accn_ref[...], 1e-12, None))
        hv_ref[0, pl.ds(i * R, R)] = (pos - neg).reshape(R)

    @pl.when(jnp.logical_and(i == NRB - 1, j == NCB - 1))
    def _hist():
        hv = hv_ref[...]
        max_val = jnp.maximum(MAX_DIST, jnp.max(hv))
        min_val = jnp.minimum(-MAX_DIST, jnp.min(hv))
        bw = (max_val - min_val) / (NBINS - 1)
        lo = jnp.floor((hv - min_val) / bw).astype(jnp.int32)
        hi = jnp.minimum(lo + 1, NBINS - 1)
        alpha = 1.0 - (hv - min_val - lo.astype(jnp.float32) * bw) / bw
        bins = jax.lax.broadcasted_iota(jnp.int32, (NBINS, N), 0)
        contrib = (jnp.where(bins == lo, alpha, 0.0)
                   + jnp.where(bins == hi, 1.0 - alpha, 0.0))
        hist = jnp.sum(contrib, axis=1, keepdims=True)
        hist = hist / (jnp.sum(hist) + 1e-6)
        pdf = hist / jnp.sum(hist)
        w = jnp.sum(jnp.where(bins <= lo, pdf, 0.0), axis=0, keepdims=True)
        out_ref[...] = (jnp.sum(hv * w) / N).reshape(1, 1)


@jax.jit
def kernel(x, targets, histogram):
    del histogram
    order = jnp.argsort(targets)
    ts = targets[order]
    xs = x[order]
    xt = xs.T
    tcol = ts.reshape(N, 1)
    trow = ts.reshape(1, N)
    rl = ts.reshape(NRB, R)
    cl = ts.reshape(NCB, C)
    loss = pl.pallas_call(
        _body,
        grid=(NRB, NCB),
        in_specs=[
            pl.BlockSpec(memory_space=pltpu.SMEM),
            pl.BlockSpec(memory_space=pltpu.SMEM),
            pl.BlockSpec(memory_space=pltpu.SMEM),
            pl.BlockSpec(memory_space=pltpu.SMEM),
            pl.BlockSpec((R, D), lambda i, j: (i, 0)),
            pl.BlockSpec((D, C), lambda i, j: (0, j)),
            pl.BlockSpec((R, 1), lambda i, j: (i, 0)),
            pl.BlockSpec((1, C), lambda i, j: (0, j)),
        ],
        out_specs=pl.BlockSpec((1, 1), lambda i, j: (0, 0)),
        out_shape=jax.ShapeDtypeStruct((1, 1), jnp.float32),
        scratch_shapes=[
            pltpu.VMEM((1, R), jnp.float32),
            pltpu.VMEM((1, R), jnp.float32),
            pltpu.VMEM((1, N), jnp.float32),
        ],
    )(rl[:, 0], rl[:, -1], cl[:, 0], cl[:, -1], xs, xt, tcol, trow)
    return loss.reshape(())


# speedup vs baseline: 4.9468x; 4.9468x over previous
import jax
import jax.numpy as jnp
from jax.experimental import pallas as pl
from jax.experimental.pallas import tpu as pltpu

N = 8192
D = 64
NBINS = 64
MAX_DIST = 2.0
ROW_BLOCK = 512
N_BLOCKS = N // ROW_BLOCK


def _body(xb_ref, xt_ref, tcol_ref, trow_ref, out_ref, hv_ref):
    i = pl.program_id(0)
    xb = xb_ref[...]
    xt = xt_ref[...]
    dot = jax.lax.dot_general(
        xb.astype(jnp.bfloat16), xt.astype(jnp.bfloat16),
        (((1,), (0,)), ((), ())),
        preferred_element_type=jnp.float32,
    )
    sq_r = jnp.sum(xb * xb, axis=1)
    h = 0.5 * jnp.sum(xt * xt, axis=0, keepdims=True)
    e = h - dot
    mask = tcol_ref[...] == trow_ref[...]
    posq = sq_r + 2.0 * jnp.max(jnp.where(mask, e, -jnp.inf), axis=1)
    negq = sq_r + 2.0 * jnp.min(jnp.where(mask, jnp.inf, e), axis=1)
    pos = jnp.sqrt(jnp.clip(posq, 1e-12, None))
    neg = jnp.sqrt(jnp.clip(negq, 1e-12, None))
    hv_ref[0, pl.ds(i * ROW_BLOCK, ROW_BLOCK)] = pos - neg

    @pl.when(i == N_BLOCKS - 1)
    def _hist():
        hv = hv_ref[...]
        max_val = jnp.maximum(MAX_DIST, jnp.max(hv))
        min_val = jnp.minimum(-MAX_DIST, jnp.min(hv))
        bw = (max_val - min_val) / (NBINS - 1)
        lo = jnp.floor((hv - min_val) / bw).astype(jnp.int32)
        hi = jnp.minimum(lo + 1, NBINS - 1)
        alpha = 1.0 - (hv - min_val - lo.astype(jnp.float32) * bw) / bw
        bins = jax.lax.broadcasted_iota(jnp.int32, (NBINS, N), 0)
        contrib = (jnp.where(bins == lo, alpha, 0.0)
                   + jnp.where(bins == hi, 1.0 - alpha, 0.0))
        hist = jnp.sum(contrib, axis=1, keepdims=True)
        hist = hist / (jnp.sum(hist) + 1e-6)
        pdf = hist / jnp.sum(hist)
        w = jnp.sum(jnp.where(bins <= lo, pdf, 0.0), axis=0, keepdims=True)
        out_ref[...] = (jnp.sum(hv * w) / N).reshape(1, 1)


@jax.jit
def kernel(x, targets, histogram):
    del histogram
    xt = x.T
    tcol = targets.reshape(N, 1)
    trow = targets.reshape(1, N)
    loss = pl.pallas_call(
        _body,
        grid=(N_BLOCKS,),
        in_specs=[
            pl.BlockSpec((ROW_BLOCK, D), lambda i: (i, 0)),
            pl.BlockSpec((D, N), lambda i: (0, 0)),
            pl.BlockSpec((ROW_BLOCK, 1), lambda i: (i, 0)),
            pl.BlockSpec((1, N), lambda i: (0, 0)),
        ],
        out_specs=pl.BlockSpec((1, 1), lambda i: (0, 0)),
        out_shape=jax.ShapeDtypeStruct((1, 1), jnp.float32),
        scratch_shapes=[pltpu.VMEM((1, N), jnp.float32)],
    )(x, xt, tcol, trow)
    return loss.reshape(())


# speedup vs baseline: 6.8055x; 1.3757x over previous
import jax
import jax.numpy as jnp
from jax.experimental import pallas as pl
from jax.experimental.pallas import tpu as pltpu

N = 8192
D = 64
NBINS = 64
MAX_DIST = 2.0
ROW_BLOCK = 512
N_BLOCKS = N // ROW_BLOCK


def _body(xb_ref, xt_ref, tcol_ref, trow_ref, out_ref, hv_ref):
    i = pl.program_id(0)
    xb = xb_ref[...]
    xt = xt_ref[...]
    dot = jax.lax.dot_general(
        xb.astype(jnp.bfloat16), xt.astype(jnp.bfloat16),
        (((1,), (0,)), ((), ())),
        preferred_element_type=jnp.float32,
    )
    sq_r = jnp.sum(xb * xb, axis=1)
    h = (0.5 * jnp.sum(xt * xt, axis=0, keepdims=True)).astype(jnp.bfloat16)
    e = h - dot.astype(jnp.bfloat16)
    mask = tcol_ref[...] == trow_ref[...]
    ninf = jnp.asarray(-jnp.inf, jnp.bfloat16)
    pinf = jnp.asarray(jnp.inf, jnp.bfloat16)
    pmax = jnp.max(jnp.where(mask, e, ninf), axis=1).astype(jnp.float32)
    nmin = jnp.min(jnp.where(mask, pinf, e), axis=1).astype(jnp.float32)
    posq = sq_r + 2.0 * pmax
    negq = sq_r + 2.0 * nmin
    pos = jnp.sqrt(jnp.clip(posq, 1e-12, None))
    neg = jnp.sqrt(jnp.clip(negq, 1e-12, None))
    hv_ref[0, pl.ds(i * ROW_BLOCK, ROW_BLOCK)] = pos - neg

    @pl.when(i == N_BLOCKS - 1)
    def _hist():
        hv = hv_ref[...]
        max_val = jnp.maximum(MAX_DIST, jnp.max(hv))
        min_val = jnp.minimum(-MAX_DIST, jnp.min(hv))
        bw = (max_val - min_val) / (NBINS - 1)
        lo = jnp.floor((hv - min_val) / bw).astype(jnp.int32)
        hi = jnp.minimum(lo + 1, NBINS - 1)
        alpha = 1.0 - (hv - min_val - lo.astype(jnp.float32) * bw) / bw
        bins = jax.lax.broadcasted_iota(jnp.int32, (NBINS, N), 0)
        contrib = (jnp.where(bins == lo, alpha, 0.0)
                   + jnp.where(bins == hi, 1.0 - alpha, 0.0))
        hist = jnp.sum(contrib, axis=1, keepdims=True)
        hist = hist / (jnp.sum(hist) + 1e-6)
        pdf = hist / jnp.sum(hist)
        w = jnp.sum(jnp.where(bins <= lo, pdf, 0.0), axis=0, keepdims=True)
        out_ref[...] = (jnp.sum(hv * w) / N).reshape(1, 1)


@jax.jit
def kernel(x, targets, histogram):
    del histogram
    xt = x.T
    tcol = targets.reshape(N, 1)
    trow = targets.reshape(1, N)
    loss = pl.pallas_call(
        _body,
        grid=(N_BLOCKS,),
        in_specs=[
            pl.BlockSpec((ROW_BLOCK, D), lambda i: (i, 0)),
            pl.BlockSpec((D, N), lambda i: (0, 0)),
            pl.BlockSpec((ROW_BLOCK, 1), lambda i: (i, 0)),
            pl.BlockSpec((1, N), lambda i: (0, 0)),
        ],
        out_specs=pl.BlockSpec((1, 1), lambda i: (0, 0)),
        out_shape=jax.ShapeDtypeStruct((1, 1), jnp.float32),
        scratch_shapes=[pltpu.VMEM((1, N), jnp.float32)],
    )(x, xt, tcol, trow)
    return loss.reshape(())


# speedup vs baseline: 7.7553x; 1.1396x over previous
import jax
import jax.numpy as jnp
from jax.experimental import pallas as pl
from jax.experimental.pallas import tpu as pltpu

N = 8192
D = 64
NBINS = 64
MAX_DIST = 2.0
ROW_BLOCK = 512
N_BLOCKS = N // ROW_BLOCK


def _body(xb_ref, xt_ref, tcol_ref, trow_ref, out_ref, hv_ref):
    i = pl.program_id(0)
    xb = xb_ref[...]
    xt = xt_ref[...]
    sq_r = jnp.sum(xb * xb, axis=1)
    h = 0.5 * jnp.sum(xt * xt, axis=0, keepdims=True)
    lhs = jnp.concatenate(
        [(-xb).astype(jnp.bfloat16),
         jnp.ones((xb.shape[0], 1), jnp.bfloat16)], axis=1)
    rhs = jnp.concatenate(
        [xt.astype(jnp.bfloat16), h.astype(jnp.bfloat16)], axis=0)
    e32 = jax.lax.dot_general(
        lhs, rhs, (((1,), (0,)), ((), ())),
        preferred_element_type=jnp.float32,
    )
    e = e32.astype(jnp.bfloat16)
    mask = tcol_ref[...] == trow_ref[...]
    ninf = jnp.asarray(-jnp.inf, jnp.bfloat16)
    pinf = jnp.asarray(jnp.inf, jnp.bfloat16)
    pmax = jnp.max(jnp.where(mask, e, ninf), axis=1).astype(jnp.float32)
    nmin = jnp.min(jnp.where(mask, pinf, e), axis=1).astype(jnp.float32)
    posq = sq_r + 2.0 * pmax
    negq = sq_r + 2.0 * nmin
    pos = jnp.sqrt(jnp.clip(posq, 1e-12, None))
    neg = jnp.sqrt(jnp.clip(negq, 1e-12, None))
    hv_ref[0, pl.ds(i * ROW_BLOCK, ROW_BLOCK)] = pos - neg

    @pl.when(i == N_BLOCKS - 1)
    def _hist():
        hv = hv_ref[...]
        max_val = jnp.maximum(MAX_DIST, jnp.max(hv))
        min_val = jnp.minimum(-MAX_DIST, jnp.min(hv))
        bw = (max_val - min_val) / (NBINS - 1)
        lo = jnp.floor((hv - min_val) / bw).astype(jnp.int32)
        hi = jnp.minimum(lo + 1, NBINS - 1)
        alpha = 1.0 - (hv - min_val - lo.astype(jnp.float32) * bw) / bw
        bins = jax.lax.broadcasted_iota(jnp.int32, (NBINS, N), 0)
        contrib = (jnp.where(bins == lo, alpha, 0.0)
                   + jnp.where(bins == hi, 1.0 - alpha, 0.0))
        hist = jnp.sum(contrib, axis=1, keepdims=True)
        hist = hist / (jnp.sum(hist) + 1e-6)
        pdf = hist / jnp.sum(hist)
        w = jnp.sum(jnp.where(bins <= lo, pdf, 0.0), axis=0, keepdims=True)
        out_ref[...] = (jnp.sum(hv * w) / N).reshape(1, 1)


@jax.jit
def kernel(x, targets, histogram):
    del histogram
    xt = x.T
    tcol = targets.reshape(N, 1)
    trow = targets.reshape(1, N)
    loss = pl.pallas_call(
        _body,
        grid=(N_BLOCKS,),
        in_specs=[
            pl.BlockSpec((ROW_BLOCK, D), lambda i: (i, 0)),
            pl.BlockSpec((D, N), lambda i: (0, 0)),
            pl.BlockSpec((ROW_BLOCK, 1), lambda i: (i, 0)),
            pl.BlockSpec((1, N), lambda i: (0, 0)),
        ],
        out_specs=pl.BlockSpec((1, 1), lambda i: (0, 0)),
        out_shape=jax.ShapeDtypeStruct((1, 1), jnp.float32),
        scratch_shapes=[pltpu.VMEM((1, N), jnp.float32)],
    )(x, xt, tcol, trow)
    return loss.reshape(())
